# baseline (speedup 1.0000x reference)
import functools

import numpy as np
import jax
import jax.numpy as jnp
from jax import lax
from jax.experimental import pallas as pl
from jax.experimental.pallas import tpu as pltpu

_B = 1024
_L1 = 1024
_NUM_FEATURES = 800
_BT = 128
_NBT = _B // _BT

_S = np.zeros((96, 3, 10), dtype=np.float32)
for _dx in range(3):
    for _j in range(10):
        _c = 10 * _j - 1 + _dx
        if 0 <= _c < 96:
            _S[_c, _dx, _j] = 1.0

_PERM = np.zeros((_NUM_FEATURES,), dtype=np.int32)
for _i in range(10):
    for _o in range(8):
        for _j in range(10):
            _PERM[_i * 80 + _o * 10 + _j] = _o * 100 + _i * 10 + _j


def _dot_t(x, w):
    return lax.dot_general(x, w, (((1,), (1,)), ((), ())),
                           preferred_element_type=jnp.float32)


def _body(img_hbm, m_ref, ftw_ref, ftb_ref, w1_ref, b1_ref, w2_ref, b2_ref,
          w3_ref, b3_ref, out_ref, xbuf, sems):
    k = pl.program_id(0)

    def start_dmas(tile, buf):
        b0 = tile * _BT
        pltpu.make_async_copy(
            img_hbm.at[pl.ds(b0, _BT), :, pl.ds(0, 2), :],
            xbuf.at[buf, :, :, pl.ds(1, 2), :],
            sems.at[buf, 0]).start()
        for i in range(1, 10):
            pltpu.make_async_copy(
                img_hbm.at[pl.ds(b0, _BT), :, pl.ds(10 * i - 1, 3), :],
                xbuf.at[buf, :, :, pl.ds(3 * i, 3), :],
                sems.at[buf, i]).start()

    def wait_dmas(tile, buf):
        b0 = tile * _BT
        pltpu.make_async_copy(
            img_hbm.at[pl.ds(b0, _BT), :, pl.ds(0, 2), :],
            xbuf.at[buf, :, :, pl.ds(1, 2), :],
            sems.at[buf, 0]).wait()
        for i in range(1, 10):
            pltpu.make_async_copy(
                img_hbm.at[pl.ds(b0, _BT), :, pl.ds(10 * i - 1, 3), :],
                xbuf.at[buf, :, :, pl.ds(3 * i, 3), :],
                sems.at[buf, i]).wait()

    buf = k % 2

    @pl.when(k == 0)
    def _():
        start_dmas(0, 0)

    wait_dmas(k, buf)

    @pl.when(k + 1 < _NBT)
    def _():
        start_dmas(k + 1, (k + 1) % 2)

    x = xbuf[buf]
    x = jnp.concatenate(
        [jnp.zeros((_BT, 3, 1, 96), jnp.float32), x[:, :, 1:, :]], axis=2)

    m = m_ref[...]
    outs = []
    for i in range(10):
        xi = x[:, :, 3 * i:3 * i + 3, :].reshape(_BT, 864)
        outs.append(jnp.dot(xi, m, preferred_element_type=jnp.float32))
    conv_x = jnp.stack(outs, axis=1).reshape(_BT, _NUM_FEATURES)

    bf = jax.nn.sigmoid(10.0 * jnp.clip(conv_x, -1.0, 1.0))
    v = jnp.where(bf > 0.5, bf, 0.0)

    feat = jnp.dot(v, ftw_ref[...], preferred_element_type=jnp.float32)
    feat = feat + ftb_ref[...]
    l0 = jnp.clip(feat, 0.0, 1.0)
    s0 = l0[:, :_L1 // 2]
    s1 = l0[:, _L1 // 2:]
    l0c = jnp.concatenate([s0 * s1, s0], axis=1) * (127.0 / 128.0)

    h = jax.nn.relu(_dot_t(l0c, w1_ref[...]) + b1_ref[...])
    h = jax.nn.relu(_dot_t(h, w2_ref[...]) + b2_ref[...])
    out_ref[...] = _dot_t(h, w3_ref[...]) + b3_ref[0, 0]


@jax.jit
def kernel(images, conv_w, ft_w, ft_b, w1, b1, w2, b2, w3, b3):
    m = jnp.einsum("ocyx,wxj->cywoj", conv_w, jnp.asarray(_S))
    m = m.reshape(864, 80)
    ftw_r = ft_w[jnp.asarray(_PERM)]

    in_specs = [
            pl.BlockSpec(memory_space=pltpu.MemorySpace.HBM),
            pl.BlockSpec((864, 80), lambda k: (0, 0)),
            pl.BlockSpec((_NUM_FEATURES, _L1), lambda k: (0, 0)),
            pl.BlockSpec((1, _L1), lambda k: (0, 0)),
            pl.BlockSpec((15, _L1), lambda k: (0, 0)),
            pl.BlockSpec((1, 15), lambda k: (0, 0)),
            pl.BlockSpec((32, 15), lambda k: (0, 0)),
            pl.BlockSpec((1, 32), lambda k: (0, 0)),
            pl.BlockSpec((128, 32), lambda k: (0, 0)),
            pl.BlockSpec(memory_space=pltpu.MemorySpace.SMEM),
    ]
    out = pl.pallas_call(
        _body,
        grid=(_NBT,),
        in_specs=in_specs,
        out_specs=pl.BlockSpec((_BT, 128), lambda k: (k, 0)),
        out_shape=jax.ShapeDtypeStruct((_B, 128), jnp.float32),
        scratch_shapes=[
            pltpu.VMEM((2, _BT, 3, 30, 96), jnp.float32),
            pltpu.SemaphoreType.DMA((2, 10)),
        ],
    )(images, m, ftw_r, ft_b.reshape(1, _L1), w1, b1.reshape(1, 15),
      w2, b2.reshape(1, 32), jnp.pad(w3, ((0, 127), (0, 0))),
      b3.reshape(1, 1))
    return out[:, :1]
